# baseline (speedup 1.0000x reference)
import functools

import jax
import jax.numpy as jnp
from jax import lax
from jax.experimental import pallas as pl
from jax.experimental.pallas import tpu as pltpu
from jax.experimental.pallas import tpu_sc as plsc

_NUM_WORKERS = 32
_BLK = 1024
_LANES = 16


def _embed_lookup(table, seq_t, v, d, h, b):
    n_bb = b // _BLK
    n_tasks = (h * n_bb) // _NUM_WORKERS
    blocks_per_slab = _BLK // 128
    mesh = plsc.VectorSubcoreMesh(core_axis_name="c", subcore_axis_name="s")

    @functools.partial(
        pl.kernel,
        mesh=mesh,
        out_type=jax.ShapeDtypeStruct((h * (d // 8) * (b // 128), 8, 128),
                                      jnp.float32),
        scratch_types=[
            *[pltpu.VMEM((_BLK,), jnp.int32) for _ in range(2)],
            *[pltpu.VMEM((_BLK, d), jnp.float32) for _ in range(2)],
            *[pltpu.VMEM((d, _BLK), jnp.float32) for _ in range(2)],
            *[pltpu.SemaphoreType.DMA for _ in range(2)],
            *[pltpu.SemaphoreType.DMA for _ in range(2)],
        ],
        compiler_params=pltpu.CompilerParams(use_tc_tiling_on_sc=False,
                                             needs_layout_passes=False),
    )
    def k(table_hbm, seq_hbm, out_hbm, *scr):
        idx_v = scr[0:2]
        rows_v = scr[2:4]
        slab_v = scr[4:6]
        gsem = scr[6:8]
        osem = scr[8:10]
        wid = lax.axis_index("s") * 2 + lax.axis_index("c")
        lane_iota = lax.iota(jnp.int32, _LANES)

        def start_gather(t, p):
            tid = wid * n_tasks + t
            hh = tid // n_bb
            bb = tid % n_bb
            pltpu.sync_copy(seq_hbm.at[hh, pl.ds(bb * _BLK, _BLK)], idx_v[p])
            pltpu.make_async_copy(table_hbm.at[idx_v[p]], rows_v[p],
                                  gsem[p]).start()

        def wait_gather(p):
            pltpu.make_async_copy(table_hbm.at[idx_v[p]], rows_v[p],
                                  gsem[p]).wait()

        def block_copies(t, p):
            tid = wid * n_tasks + t
            hh = tid // n_bb
            bb = tid % n_bb
            copies = []
            for tr in range(d // 8):
                for j in range(blocks_per_slab):
                    m = (hh * (d // 8) + tr) * (b // 128) \
                        + bb * blocks_per_slab + j
                    copies.append(pltpu.make_async_copy(
                        slab_v[p].at[pl.ds(tr * 8, 8), pl.ds(j * 128, 128)],
                        out_hbm.at[m], osem[p]))
            return copies

        def transpose(p):
            for j in range(_BLK):
                row = rows_v[p][j, :]
                plsc.store_scatter(
                    slab_v[p], [lane_iota, jnp.full((_LANES,), j, jnp.int32)],
                    row)

        start_gather(0, 0)

        def do_task(t, p, drain, start_next):
            if drain is True:
                for c in block_copies(t - 2, p):
                    c.wait()
            elif drain is not None:
                @pl.when(drain)
                def _():
                    for c in block_copies(t - 2, p):
                        c.wait()
            wait_gather(p)
            if start_next:
                start_gather(t + 1, 1 - p)
            transpose(p)
            for c in block_copies(t, p):
                c.start()

        assert n_tasks % 2 == 1
        n_rounds = n_tasks // 2

        def round_body(g, carry):
            for p in range(2):
                do_task(g * 2 + p, p, g >= 1, True)
            return carry

        lax.fori_loop(0, n_rounds, round_body, 0)
        do_task(n_tasks - 1, (n_tasks - 1) % 2, True, False)
        for tl in (n_tasks - 2, n_tasks - 1):
            for c in block_copies(tl, tl % 2):
                c.wait()

    return k(table, seq_t)


def kernel(seq, table):
    b, h = seq.shape
    v, d = table.shape
    seq_t = seq.T.astype(jnp.int32)
    out = _embed_lookup(table, seq_t, v, d, h, b)
    out5 = out.reshape(h, d // 8, b // 128, 8, 128)
    return out5.transpose(2, 4, 0, 1, 3).reshape(b, h, d)


# speedup vs baseline: 1.1120x; 1.1120x over previous
import functools

import jax
import jax.numpy as jnp
from jax import lax
from jax.experimental import pallas as pl
from jax.experimental.pallas import tpu as pltpu
from jax.experimental.pallas import tpu_sc as plsc

_NUM_WORKERS = 32
_BLK = 1024
_LANES = 16


def _embed_lookup(table, seq_t, v, d, h, b):
    n_bb = b // _BLK
    n_tasks = (h * n_bb) // _NUM_WORKERS
    blocks_per_slab = _BLK // 128
    mesh = plsc.VectorSubcoreMesh(core_axis_name="c", subcore_axis_name="s")

    @functools.partial(
        pl.kernel,
        mesh=mesh,
        out_type=jax.ShapeDtypeStruct((h * (d // 8) * (b // 128), 8, 128),
                                      jnp.float32),
        scratch_types=[
            *[pltpu.VMEM((_BLK,), jnp.int32) for _ in range(2)],
            *[pltpu.VMEM((_BLK, d), jnp.float32) for _ in range(2)],
            *[pltpu.VMEM((d, _BLK), jnp.float32) for _ in range(2)],
            *[pltpu.SemaphoreType.DMA for _ in range(2)],
            *[pltpu.SemaphoreType.DMA for _ in range(2)],
        ],
        compiler_params=pltpu.CompilerParams(use_tc_tiling_on_sc=False,
                                             needs_layout_passes=False),
    )
    def k(table_hbm, seq_hbm, out_hbm, *scr):
        idx_v = scr[0:2]
        rows_v = scr[2:4]
        slab_v = scr[4:6]
        gsem = scr[6:8]
        osem = scr[8:10]
        wid = lax.axis_index("s") * 2 + lax.axis_index("c")
        lane_iota = lax.iota(jnp.int32, _LANES)

        def start_gather(t, p):
            tid = wid * n_tasks + t
            hh = tid // n_bb
            bb = tid % n_bb
            pltpu.sync_copy(seq_hbm.at[hh, pl.ds(bb * _BLK, _BLK)], idx_v[p])
            pltpu.make_async_copy(table_hbm.at[idx_v[p]], rows_v[p],
                                  gsem[p]).start()

        def wait_gather(p):
            pltpu.make_async_copy(table_hbm.at[idx_v[p]], rows_v[p],
                                  gsem[p]).wait()

        def block_copies(t, p):
            tid = wid * n_tasks + t
            hh = tid // n_bb
            bb = tid % n_bb
            copies = []
            for tr in range(d // 8):
                for j in range(blocks_per_slab):
                    m = (hh * (d // 8) + tr) * (b // 128) \
                        + bb * blocks_per_slab + j
                    copies.append(pltpu.make_async_copy(
                        slab_v[p].at[pl.ds(tr * 8, 8), pl.ds(j * 128, 128)],
                        out_hbm.at[m], osem[p]))
            return copies

        def transpose(p):
            @plsc.parallel_loop(0, _BLK, 1, unroll=16)
            def _(j):
                row = rows_v[p][j, :]
                plsc.store_scatter(
                    slab_v[p],
                    [lane_iota, jnp.full((_LANES,), 0, jnp.int32) + j],
                    row)

        start_gather(0, 0)

        def do_task(t, p, drain, start_next):
            if drain is True:
                for c in block_copies(t - 2, p):
                    c.wait()
            elif drain is not None:
                @pl.when(drain)
                def _():
                    for c in block_copies(t - 2, p):
                        c.wait()
            wait_gather(p)
            if start_next:
                start_gather(t + 1, 1 - p)
            transpose(p)
            for c in block_copies(t, p):
                c.start()

        assert n_tasks % 2 == 1
        n_rounds = n_tasks // 2

        def round_body(g, carry):
            for p in range(2):
                do_task(g * 2 + p, p, g >= 1, True)
            return carry

        lax.fori_loop(0, n_rounds, round_body, 0)
        do_task(n_tasks - 1, (n_tasks - 1) % 2, True, False)
        for tl in (n_tasks - 2, n_tasks - 1):
            for c in block_copies(tl, tl % 2):
                c.wait()

    return k(table, seq_t)


def kernel(seq, table):
    b, h = seq.shape
    v, d = table.shape
    seq_t = seq.T.astype(jnp.int32)
    out = _embed_lookup(table, seq_t, v, d, h, b)
    out5 = out.reshape(h, d // 8, b // 128, 8, 128)
    return out5.transpose(2, 4, 0, 1, 3).reshape(b, h, d)


# speedup vs baseline: 1.1174x; 1.0048x over previous
import functools

import jax
import jax.numpy as jnp
from jax import lax
from jax.experimental import pallas as pl
from jax.experimental.pallas import tpu as pltpu
from jax.experimental.pallas import tpu_sc as plsc

_NUM_WORKERS = 32
_BLK = 1024
_LANES = 16


def _embed_lookup(table, seq_t, v, d, h, b):
    n_bb = b // _BLK
    n_tasks = (h * n_bb) // _NUM_WORKERS
    blocks_per_slab = _BLK // 128
    mesh = plsc.VectorSubcoreMesh(core_axis_name="c", subcore_axis_name="s")

    @functools.partial(
        pl.kernel,
        mesh=mesh,
        out_type=jax.ShapeDtypeStruct((h * (d // 8) * (b // 128), 8, 128),
                                      jnp.float32),
        scratch_types=[
            *[pltpu.VMEM((_BLK,), jnp.int32) for _ in range(2)],
            *[pltpu.VMEM((_BLK, d), jnp.float32) for _ in range(2)],
            *[pltpu.VMEM((d, _BLK), jnp.float32) for _ in range(2)],
            *[pltpu.SemaphoreType.DMA for _ in range(2)],
            *[pltpu.SemaphoreType.DMA for _ in range(2)],
        ],
        compiler_params=pltpu.CompilerParams(use_tc_tiling_on_sc=False,
                                             needs_layout_passes=False),
    )
    def k(table_hbm, seq_hbm, out_hbm, *scr):
        idx_v = scr[0:2]
        rows_v = scr[2:4]
        slab_v = scr[4:6]
        gsem = scr[6:8]
        osem = scr[8:10]
        wid = lax.axis_index("s") * 2 + lax.axis_index("c")
        lane_iota = lax.iota(jnp.int32, _LANES)

        def start_gather(t, p):
            tid = wid * n_tasks + t
            hh = tid // n_bb
            bb = tid % n_bb
            pltpu.sync_copy(seq_hbm.at[hh, pl.ds(bb * _BLK, _BLK)], idx_v[p])
            pltpu.make_async_copy(table_hbm.at[idx_v[p]], rows_v[p],
                                  gsem[p]).start()

        def wait_gather(p):
            pltpu.make_async_copy(table_hbm.at[idx_v[p]], rows_v[p],
                                  gsem[p]).wait()

        def block_copies(t, p):
            tid = wid * n_tasks + t
            hh = tid // n_bb
            bb = tid % n_bb
            copies = []
            for tr in range(d // 8):
                for j in range(blocks_per_slab):
                    m = (hh * (d // 8) + tr) * (b // 128) \
                        + bb * blocks_per_slab + j
                    copies.append(pltpu.make_async_copy(
                        slab_v[p].at[pl.ds(tr * 8, 8), pl.ds(j * 128, 128)],
                        out_hbm.at[m], osem[p]))
            return copies

        def transpose(p):
            @plsc.parallel_loop(0, _BLK, 1, unroll=32)
            def _(j):
                row = rows_v[p][j, :]
                plsc.store_scatter(
                    slab_v[p],
                    [lane_iota, jnp.full((_LANES,), 0, jnp.int32) + j],
                    row)

        start_gather(0, 0)

        def do_task(t, p, drain, start_next):
            if drain is True:
                for c in block_copies(t - 2, p):
                    c.wait()
            elif drain is not None:
                @pl.when(drain)
                def _():
                    for c in block_copies(t - 2, p):
                        c.wait()
            wait_gather(p)
            if start_next:
                start_gather(t + 1, 1 - p)
            transpose(p)
            for c in block_copies(t, p):
                c.start()

        assert n_tasks % 2 == 1
        n_rounds = n_tasks // 2

        def round_body(g, carry):
            for p in range(2):
                do_task(g * 2 + p, p, g >= 1, True)
            return carry

        lax.fori_loop(0, n_rounds, round_body, 0)
        do_task(n_tasks - 1, (n_tasks - 1) % 2, True, False)
        for tl in (n_tasks - 2, n_tasks - 1):
            for c in block_copies(tl, tl % 2):
                c.wait()

    return k(table, seq_t)


def kernel(seq, table):
    b, h = seq.shape
    v, d = table.shape
    seq_t = seq.T.astype(jnp.int32)
    out = _embed_lookup(table, seq_t, v, d, h, b)
    out5 = out.reshape(h, d // 8, b // 128, 8, 128)
    return out5.transpose(2, 4, 0, 1, 3).reshape(b, h, d)
